# baseline (speedup 1.0000x reference)
import functools

import jax
import jax.numpy as jnp
from jax import lax
from jax.experimental import pallas as pl
from jax.experimental.pallas import tpu as pltpu
from jax.experimental.pallas import tpu_sc as plsc

VOCAB = 30522
HIDDEN = 768
MAX_POS = 512
TYPE_VOCAB = 2
BATCH = 64
SEQ = 512
EPS = 1e-12

L = 16
NJ = HIDDEN // L
NW = 32
POS_PER_W = SEQ // NW


def _rsqrt_vec(x):
    i = plsc.bitcast(x, jnp.int32)
    i = jnp.int32(0x5F3759DF) - lax.shift_right_logical(i, 1)
    y = plsc.bitcast(i, jnp.float32)
    half = x * 0.5
    for _ in range(3):
        y = y * (1.5 - half * y * y)
    return y


def _body(ids_hbm, tt_hbm, word_hbm, pos_hbm, type_hbm, gamma_hbm, beta_hbm,
          out_hbm, idx_v, tt_v, pbuf, rbuf, type_v, dbuf, gbuf, bbuf, sem):
    wid = lax.axis_index("s") * 2 + lax.axis_index("c")
    col0 = wid * POS_PER_W

    pltpu.sync_copy(ids_hbm.at[pl.ds(wid * BATCH * POS_PER_W, BATCH * POS_PER_W)], idx_v)
    pltpu.sync_copy(tt_hbm.at[pl.ds(wid * BATCH * POS_PER_W, BATCH * POS_PER_W)], tt_v)
    pltpu.sync_copy(pos_hbm.at[pl.ds(col0, POS_PER_W)], pbuf)
    pltpu.sync_copy(type_hbm, type_v)
    pltpu.sync_copy(gamma_hbm, gbuf)
    pltpu.sync_copy(beta_hbm, bbuf)

    for j in range(NJ):
        d = pl.ds(j * L, L)
        dbuf[d] = type_v[1, d] - type_v[0, d]

    def build_pos(t, _):
        for j in range(NJ):
            d = pl.ds(j * L, L)
            pbuf[t, d] = pbuf[t, d] + type_v[0, d]
        return _

    lax.fori_loop(0, POS_PER_W, build_pos, None)

    def per_batch(b, _):
        pltpu.async_copy(word_hbm.at[idx_v.at[pl.ds(b * POS_PER_W, POS_PER_W)]],
                         rbuf, sem).wait()

        ttrow = tt_v[pl.ds(b * POS_PER_W, POS_PER_W)]

        def per_token(t, _):
            ttb = lax.gather(
                ttrow, jnp.full((L, 1), t, jnp.int32),
                lax.GatherDimensionNumbers(offset_dims=(),
                                           collapsed_slice_dims=(0,),
                                           start_index_map=(0,)),
                (1,), mode=lax.GatherScatterMode.PROMISE_IN_BOUNDS)
            acc = jnp.zeros((L,), jnp.float32)
            acc2 = jnp.zeros((L,), jnp.float32)
            for j in range(NJ):
                d = pl.ds(j * L, L)
                v = rbuf[t, d] + (pbuf[t, d] + ttb * dbuf[d])
                rbuf[t, d] = v
                acc = acc + v
                acc2 = acc2 + v * v
            s1 = jnp.sum(acc)
            s2 = jnp.sum(acc2)
            mean = s1 * (1.0 / HIDDEN)
            var = s2 * (1.0 / HIDDEN) - mean * mean
            mean_v = jnp.full((L,), mean, jnp.float32)
            rstd_v = _rsqrt_vec(jnp.full((L,), var + EPS, jnp.float32))
            for j in range(NJ):
                d = pl.ds(j * L, L)
                y = (rbuf[t, d] - mean_v) * rstd_v
                rbuf[t, d] = y * gbuf[d] + bbuf[d]
            return _

        lax.fori_loop(0, POS_PER_W, per_token, None)
        pltpu.sync_copy(rbuf, out_hbm.at[pl.ds(b * SEQ + col0, POS_PER_W)])
        return _

    lax.fori_loop(0, BATCH, per_batch, None)


@jax.jit
def _run(input_ids, token_type_ids, word_emb, pos_emb, type_emb, gamma, beta):
    mesh = plsc.VectorSubcoreMesh(core_axis_name="c", subcore_axis_name="s",
                                  num_cores=2, num_subcores=16)
    out = pl.kernel(
        _body,
        out_type=jax.ShapeDtypeStruct((BATCH * SEQ, HIDDEN), jnp.float32),
        mesh=mesh,
        compiler_params=pltpu.CompilerParams(needs_layout_passes=False),
        scratch_types=[
            pltpu.VMEM((BATCH * POS_PER_W,), jnp.int32),
            pltpu.VMEM((BATCH * POS_PER_W,), jnp.float32),
            pltpu.VMEM((POS_PER_W, HIDDEN), jnp.float32),
            pltpu.VMEM((POS_PER_W, HIDDEN), jnp.float32),
            pltpu.VMEM((TYPE_VOCAB, HIDDEN), jnp.float32),
            pltpu.VMEM((HIDDEN,), jnp.float32),
            pltpu.VMEM((HIDDEN,), jnp.float32),
            pltpu.VMEM((HIDDEN,), jnp.float32),
            pltpu.SemaphoreType.DMA,
        ],
    )(input_ids, token_type_ids, word_emb, pos_emb, type_emb, gamma, beta)
    return out.reshape(BATCH, SEQ, HIDDEN)


def _worker_layout(x):
    return x.reshape(BATCH, NW, POS_PER_W).transpose(1, 0, 2).reshape(-1)


def kernel(input_ids, token_type_ids, word_emb, pos_emb, type_emb, gamma, beta):
    return _run(_worker_layout(input_ids.astype(jnp.int32)),
                _worker_layout(token_type_ids.astype(jnp.float32)),
                word_emb, pos_emb, type_emb, gamma, beta)
